# baseline (speedup 1.0000x reference)
import functools

import jax
import jax.numpy as jnp
from jax import lax
from jax.experimental import pallas as pl
from jax.experimental.pallas import tpu as pltpu
from jax.experimental.pallas import tpu_sc as plsc

N = 10000
E = 320000
D = 128
NC, NS = 2, 16
NW = NC * NS
CH = 112
NCHUNK = -(-(E // NW) // CH)
PW = NCHUNK * CH
EP = NW * PW
NP = 10240
RPT = NP // NS
NRB = 3
NSL = 5

_mesh = plsc.VectorSubcoreMesh(core_axis_name="c", subcore_axis_name="s")


@functools.partial(
    pl.kernel,
    out_type=jax.ShapeDtypeStruct((NC, NP, D), jnp.float32),
    mesh=_mesh,
    scratch_types=[
        pltpu.VMEM((NSL, 2, CH), jnp.int32),
        pltpu.VMEM((NSL, CH), jnp.float32),
        pltpu.VMEM((NRB, CH, D), jnp.float32),
        pltpu.VMEM_SHARED((NP, D), jnp.float32),
        pltpu.SemaphoreType.DMA((NSL,)),
        pltpu.SemaphoreType.DMA((NRB,)),
        pltpu.SemaphoreType.DMA((NRB,)),
    ],
)
def _sc_aggregate(x_hbm, meta_hbm, w_hbm, part_hbm,
                  slots, slots_w, rows, acc_sh, isem, gsem, ssem):
    cid = lax.axis_index("c")
    sid = lax.axis_index("s")
    wid = sid * NC + cid

    def _fill_slot(c, s):
        pltpu.async_copy(meta_hbm.at[wid, c], slots.at[s], isem.at[s])
        pltpu.async_copy(w_hbm.at[wid, c], slots_w.at[s], isem.at[s])

    def _wait_slot(c, s):
        pltpu.make_async_copy(
            meta_hbm.at[wid, c], slots.at[s], isem.at[s]).wait()
        pltpu.make_async_copy(
            w_hbm.at[wid, c], slots_w.at[s], isem.at[s]).wait()

    for j in range(4):
        _fill_slot(j, j)
    for j in range(2):
        _wait_slot(j, j)
        pltpu.async_copy(x_hbm.at[slots.at[j, 0]], rows.at[j], gsem.at[j])

    def zero_body(e, carry):
        for k in range(D // 16):
            rows[2, e, pl.ds(k * 16, 16)] = jnp.zeros((16,), jnp.float32)
        return carry
    lax.fori_loop(0, CH, zero_body, 0)
    for j in range(5):
        pltpu.sync_copy(rows.at[2],
                        acc_sh.at[pl.ds(sid * RPT + j * CH, CH)])
    pltpu.sync_copy(rows.at[2, pl.ds(0, RPT - 5 * CH)],
                    acc_sh.at[pl.ds(sid * RPT + 5 * CH, RPT - 5 * CH)])
    plsc.subcore_barrier()

    def body(c, carry):
        rb = lax.rem(c, NRB)
        b2 = lax.rem(c + 2, NRB)
        s = lax.rem(c, NSL)

        pltpu.make_async_copy(
            x_hbm.at[slots.at[s, 0]], rows.at[rb], gsem.at[rb]).wait()

        def scale_body(g, inner):
            w16 = slots_w[s, pl.ds(g * 16, 16)]
            for e in range(16):
                w = lax.gather(
                    w16, jnp.full((16, 1), e, jnp.int32),
                    lax.GatherDimensionNumbers(
                        offset_dims=(), collapsed_slice_dims=(0,),
                        start_index_map=(0,)),
                    slice_sizes=(1,),
                    mode=lax.GatherScatterMode.PROMISE_IN_BOUNDS)
                row = g * 16 + e
                for k in range(D // 16):
                    sl = pl.ds(k * 16, 16)
                    rows[rb, row, sl] = rows[rb, row, sl] * w
            return inner
        lax.fori_loop(0, CH // 16, scale_body, 0)

        pltpu.async_copy(
            rows.at[rb], acc_sh.at[slots.at[s, 1]], ssem.at[rb], add=True)

        @pl.when(c + 2 < NCHUNK)
        def _():
            @pl.when(c >= 1)
            def _():
                s1 = lax.rem(c - 1, NSL)
                pltpu.make_async_copy(
                    rows.at[b2], acc_sh.at[slots.at[s1, 1]],
                    ssem.at[b2]).wait()

            @pl.when(c + 4 < NCHUNK)
            def _():
                _fill_slot(c + 4, lax.rem(c + 4, NSL))

            s2 = lax.rem(c + 2, NSL)
            _wait_slot(c + 2, s2)
            pltpu.async_copy(
                x_hbm.at[slots.at[s2, 0]], rows.at[b2], gsem.at[b2])
        return carry
    lax.fori_loop(0, NCHUNK, body, 0)

    for c in (NCHUNK - 3, NCHUNK - 2, NCHUNK - 1):
        pltpu.make_async_copy(
            rows.at[c % NRB], acc_sh.at[slots.at[c % NSL, 1]],
            ssem.at[c % NRB]).wait()

    plsc.subcore_barrier()
    pltpu.sync_copy(acc_sh.at[pl.ds(sid * RPT, RPT)],
                    part_hbm.at[cid, pl.ds(sid * RPT, RPT)])


_TC_BLK = 1000


def _tc_body(p_ref, w_ref, b_ref, o_ref):
    s = p_ref[0] + p_ref[1]
    t = lax.dot_general(s, w_ref[...], (((1,), (0,)), ((), ())),
                        preferred_element_type=jnp.float32)
    o_ref[...] = jnp.maximum(t + b_ref[...], 0.0)


_tc_finish = pl.pallas_call(
    _tc_body,
    grid=(N // _TC_BLK,),
    in_specs=[
        pl.BlockSpec((NC, _TC_BLK, D), lambda i: (0, i, 0)),
        pl.BlockSpec((D, D), lambda i: (0, 0)),
        pl.BlockSpec((1, D), lambda i: (0, 0)),
    ],
    out_specs=pl.BlockSpec((_TC_BLK, D), lambda i: (i, 0)),
    out_shape=jax.ShapeDtypeStruct((N, D), jnp.float32),
)


def kernel(node_features, edge_index, edge_weight, kernel, bias):
    dst = edge_index[0].astype(jnp.int32)
    src = edge_index[1].astype(jnp.int32)
    pad = EP - E
    src_p = jnp.concatenate([src, jnp.zeros((pad,), jnp.int32)]).reshape(NW, NCHUNK, CH)
    dst_p = jnp.concatenate([dst, jnp.zeros((pad,), jnp.int32)]).reshape(NW, NCHUNK, CH)
    w_p = jnp.concatenate(
        [edge_weight.astype(jnp.float32), jnp.zeros((pad,), jnp.float32)]
    ).reshape(NW, NCHUNK, CH)
    meta = jnp.stack([src_p, dst_p], axis=2)
    part = _sc_aggregate(node_features, meta, w_p)
    return _tc_finish(part, kernel, bias.reshape(1, D))
